# baseline (speedup 1.0000x reference)
import functools

import jax
import jax.numpy as jnp
from jax import lax
from jax.experimental import pallas as pl
from jax.experimental.pallas import tpu as pltpu
from jax.experimental.pallas import tpu_sc as plsc

_RADIUS2 = 0.25 * 0.25
_NSAMPLE = 32
_STRIDE = 40
_B, _N, _K, _M, _C = 4, 8192, 2048, 4096, 64
_NC, _NSUB, _L = 2, 16, 16
_NW = _NC * _NSUB
_QPW = _M // _NW
_NG = _QPW // _L
_QW = (3 + _C) * _NSAMPLE

_PARAMS = pltpu.CompilerParams(
    needs_layout_passes=False, use_tc_tiling_on_sc=False)
_MESH = plsc.VectorSubcoreMesh(
    core_axis_name="c", subcore_axis_name="s", num_cores=_NC,
    num_subcores=_NSUB)


def _fused_body(xyzp, ncpx, ncpy, ncpz, indt, featt, out,
                xq, yq, zq, bq, kq, cidx, bbs, cxs, cys, czs, idxb,
                r0, r1, r2, r3, r4, r5, o0, o1, o2, o3, o4, o5,
                insem, gs0, gs1, gs2, gs3, gs4, gs5,
                os0, os1, os2, os3, os4, os5):
    wid = lax.axis_index("s") * _NC + lax.axis_index("c")
    qoff = wid * _QPW

    _BN = _B * _N
    stage = [
        (xyzp.at[pl.ds(0 * _BN, _BN)], xq),
        (xyzp.at[pl.ds(1 * _BN, _BN)], yq),
        (xyzp.at[pl.ds(2 * _BN, _BN)], zq),
        (indt.at[pl.ds(qoff, _QPW)], bq),
        (indt.at[pl.ds(_M + qoff, _QPW)], kq),
    ]
    cps = [pltpu.async_copy(src, dst, insem) for src, dst in stage]
    for cp in cps:
        cp.wait()

    lanes = jnp.arange(_L, dtype=jnp.int32)
    lhi = lanes + _L
    zero16 = jnp.zeros((_L,), jnp.int32)

    for gi in range(_NG):
        qv = gi * _L + lanes
        bvec = plsc.load_gather(bq, [qv])
        kvec = plsc.load_gather(kq, [qv])
        bbs[pl.ds(gi * _L, _L)] = bvec * _N
        cidx[pl.ds(gi * _L, _L)] = bvec * _K + kvec
    pltpu.async_copy(ncpx.at[cidx], cxs, insem)
    pltpu.async_copy(ncpy.at[cidx], cys, insem)
    ccp = pltpu.async_copy(ncpz.at[cidx], czs, insem)
    pltpu.make_async_copy(ncpx.at[cidx], cxs, insem).wait()
    pltpu.make_async_copy(ncpy.at[cidx], cys, insem).wait()
    ccp.wait()

    def scan_query(q):
        qsp = jnp.full((_L,), q, jnp.int32)
        bbasev = plsc.load_gather(bbs, [qsp])
        cx = plsc.load_gather(cxs, [qsp])
        cy = plsc.load_gather(cys, [qsp])
        cz = plsc.load_gather(czs, [qsp])
        qb = q * _STRIDE
        idxb[pl.ds(qb, _L)] = bbasev
        qbv = jnp.full((_L,), qb, jnp.int32)
        bpl = bbasev + lanes
        ones = zero16 + 1
        qb31 = jnp.full((_L,), qb + _NSAMPLE - 1, jnp.int32)

        @plsc.parallel_loop(0, _N // _L, unroll=4, carry=qbv - 1)
        def omq(i, om):
            siv = i * _L
            pv = bpl + siv
            px = plsc.load_gather(xq, [pv])
            py = plsc.load_gather(yq, [pv])
            pz = plsc.load_gather(zq, [pv])
            dx = px - cx
            dy = py - cy
            dz = pz - cz
            d2 = (dx * dx + dy * dy) + dz * dz
            inb = d2 < _RADIUS2
            rank = plsc.cumsum(ones, mask=inb)
            pc = plsc.all_reduce_population_count(inb)
            plsc.store_scatter(idxb, [om + rank], pv, mask=inb)
            return jnp.minimum(om + pc, qb31)

        offm1 = omq - qbv

        raw0 = idxb[pl.ds(qb, _L)]
        raw1 = idxb[pl.ds(qb + _L, _L)]
        first = plsc.load_gather(idxb, [qbv])
        sel0 = jnp.where(lanes <= offm1, raw0, first)
        sel1 = jnp.where(lhi <= offm1, raw1, first)
        idxb[pl.ds(qb, _L)] = sel0
        idxb[pl.ds(qb + _L, _L)] = sel1

    _RING = 6
    rows = (r0, r1, r2, r3, r4, r5)
    outs = (o0, o1, o2, o3, o4, o5)
    gsems = (gs0, gs1, gs2, gs3, gs4, gs5)
    osems = (os0, os1, os2, os3, os4, os5)

    sa = lanes + 3 * _NSAMPLE
    sb = lhi + 3 * _NSAMPLE

    def gqs(q):
        return idxb.at[pl.ds(q * _STRIDE, _NSAMPLE)]

    def outsl(q):
        return out.at[pl.ds((qoff + q) * _QW, _QW)]

    def fill(q, rbuf, obuf):
        qsp = jnp.full((_L,), q, jnp.int32)
        cx = plsc.load_gather(cxs, [qsp])
        cy = plsc.load_gather(cys, [qsp])
        cz = plsc.load_gather(czs, [qsp])
        qb = q * _STRIDE
        for h in range(2):
            g = idxb[pl.ds(qb + h * _L, _L)]
            px = plsc.load_gather(xq, [g])
            py = plsc.load_gather(yq, [g])
            pz = plsc.load_gather(zq, [g])
            obuf[pl.ds(h * _L, _L)] = px - cx
            obuf[pl.ds(_NSAMPLE + h * _L, _L)] = py - cy
            obuf[pl.ds(2 * _NSAMPLE + h * _L, _L)] = pz - cz

        @plsc.parallel_loop(0, _C, unroll=4)
        def _(k):
            cj = (lanes + k) & (_C - 1)
            c32 = cj * _NSAMPLE
            va = plsc.load_gather(rbuf, [lanes, cj])
            vb = plsc.load_gather(rbuf, [lhi, cj])
            plsc.store_scatter(obuf, [sa + c32], va)
            plsc.store_scatter(obuf, [sb + c32], vb)

    def emit(qq, b):
        pltpu.make_async_copy(featt.at[gqs(qq)], rows[b], gsems[b]).wait()

        @pl.when(qq >= _RING)
        def _():
            pltpu.make_async_copy(outs[b], outsl(qq - _RING), osems[b]).wait()

        fill(qq, rows[b], outs[b])
        pltpu.async_copy(outs[b], outsl(qq), osems[b])

    _MAIN = (_QPW // _RING) * _RING

    @pl.loop(0, _MAIN, step=_RING)
    def _(q):
        for b in range(_RING):
            qq = q + b
            scan_query(qq)

            @pl.when(qq >= _RING)
            def _():
                emit(qq - _RING, b)

            pltpu.async_copy(featt.at[gqs(qq)], rows[b], gsems[b])

    for qq in range(_MAIN, _QPW):
        b = qq % _RING
        scan_query(qq)
        emit(qq - _RING, b)
        pltpu.async_copy(featt.at[gqs(qq)], rows[b], gsems[b])

    for qq in range(_QPW - _RING, _QPW):
        emit(qq, qq % _RING)
    for qq in range(_QPW - _RING, _QPW):
        b = qq % _RING
        pltpu.make_async_copy(outs[b], outsl(qq), osems[b]).wait()


_fused = functools.partial(
    pl.kernel,
    out_type=jax.ShapeDtypeStruct((_M * _QW,), jnp.float32),
    mesh=_MESH,
    compiler_params=_PARAMS,
    scratch_types=[
        pltpu.VMEM((_B * _N,), jnp.float32),
        pltpu.VMEM((_B * _N,), jnp.float32),
        pltpu.VMEM((_B * _N,), jnp.float32),
        pltpu.VMEM((_QPW,), jnp.int32),
        pltpu.VMEM((_QPW,), jnp.int32),
        pltpu.VMEM((_QPW,), jnp.int32),
        pltpu.VMEM((_QPW,), jnp.int32),
        pltpu.VMEM((_QPW,), jnp.float32),
        pltpu.VMEM((_QPW,), jnp.float32),
        pltpu.VMEM((_QPW,), jnp.float32),
        pltpu.VMEM((_QPW * _STRIDE + _L,), jnp.int32),
        pltpu.VMEM((_NSAMPLE, _C), jnp.float32),
        pltpu.VMEM((_NSAMPLE, _C), jnp.float32),
        pltpu.VMEM((_NSAMPLE, _C), jnp.float32),
        pltpu.VMEM((_NSAMPLE, _C), jnp.float32),
        pltpu.VMEM((_NSAMPLE, _C), jnp.float32),
        pltpu.VMEM((_NSAMPLE, _C), jnp.float32),
        pltpu.VMEM((_QW,), jnp.float32),
        pltpu.VMEM((_QW,), jnp.float32),
        pltpu.VMEM((_QW,), jnp.float32),
        pltpu.VMEM((_QW,), jnp.float32),
        pltpu.VMEM((_QW,), jnp.float32),
        pltpu.VMEM((_QW,), jnp.float32),
        pltpu.SemaphoreType.DMA,
    ] + [pltpu.SemaphoreType.DMA] * 12,
)(_fused_body)


def kernel(xyz, new_xyz, indices, features):
    xyzp = jnp.transpose(xyz, (2, 0, 1)).reshape(3 * _B * _N)
    ncp = jnp.transpose(new_xyz, (2, 0, 1)).reshape(3, _B * _K)
    indt = jnp.transpose(indices, (1, 0)).reshape(2 * _M)
    featt = jnp.transpose(features, (0, 2, 1)).reshape(_B * _N, _C)
    res = _fused(xyzp, ncp[0], ncp[1], ncp[2], indt, featt)
    return res.reshape(_M, 3 + _C, _NSAMPLE)


# speedup vs baseline: 1.0151x; 1.0151x over previous
import functools

import jax
import jax.numpy as jnp
from jax import lax
from jax.experimental import pallas as pl
from jax.experimental.pallas import tpu as pltpu
from jax.experimental.pallas import tpu_sc as plsc

_RADIUS2 = 0.25 * 0.25
_NSAMPLE = 32
_STRIDE = 40
_B, _N, _K, _M, _C = 4, 8192, 2048, 4096, 64
_NC, _NSUB, _L = 2, 16, 16
_NW = _NC * _NSUB
_QPW = _M // _NW
_NG = _QPW // _L
_QW = (3 + _C) * _NSAMPLE

_PARAMS = pltpu.CompilerParams(
    needs_layout_passes=False, use_tc_tiling_on_sc=False)
_MESH = plsc.VectorSubcoreMesh(
    core_axis_name="c", subcore_axis_name="s", num_cores=_NC,
    num_subcores=_NSUB)


def _fused_body(xyzp, ncpx, ncpy, ncpz, indt, featt, out,
                xq, yq, zq, bq, kq, cidx, bbs, cxs, cys, czs, idxb,
                r0, r1, r2, r3, r4, r5, o0, o1, o2, o3, o4, o5,
                insem, gs0, gs1, gs2, gs3, gs4, gs5,
                os0, os1, os2, os3, os4, os5):
    wid = lax.axis_index("s") * _NC + lax.axis_index("c")
    qoff = wid * _QPW

    _BN = _B * _N
    stage = [
        (xyzp.at[pl.ds(0 * _BN, _BN)], xq),
        (xyzp.at[pl.ds(1 * _BN, _BN)], yq),
        (xyzp.at[pl.ds(2 * _BN, _BN)], zq),
        (indt.at[pl.ds(qoff, _QPW)], bq),
        (indt.at[pl.ds(_M + qoff, _QPW)], kq),
    ]
    cps = [pltpu.async_copy(src, dst, insem) for src, dst in stage]
    for cp in cps:
        cp.wait()

    lanes = jnp.arange(_L, dtype=jnp.int32)
    lhi = lanes + _L
    zero16 = jnp.zeros((_L,), jnp.int32)

    for gi in range(_NG):
        qv = gi * _L + lanes
        bvec = plsc.load_gather(bq, [qv])
        kvec = plsc.load_gather(kq, [qv])
        bbs[pl.ds(gi * _L, _L)] = bvec * _N
        cidx[pl.ds(gi * _L, _L)] = bvec * _K + kvec
    pltpu.async_copy(ncpx.at[cidx], cxs, insem)
    pltpu.async_copy(ncpy.at[cidx], cys, insem)
    ccp = pltpu.async_copy(ncpz.at[cidx], czs, insem)
    pltpu.make_async_copy(ncpx.at[cidx], cxs, insem).wait()
    pltpu.make_async_copy(ncpy.at[cidx], cys, insem).wait()
    ccp.wait()

    def scan_query(q):
        qsp = jnp.full((_L,), q, jnp.int32)
        bbasev = plsc.load_gather(bbs, [qsp])
        cx = plsc.load_gather(cxs, [qsp])
        cy = plsc.load_gather(cys, [qsp])
        cz = plsc.load_gather(czs, [qsp])
        qb = q * _STRIDE
        idxb[pl.ds(qb, _L)] = bbasev
        qbv = jnp.full((_L,), qb, jnp.int32)
        bpl = bbasev + lanes
        ones = zero16 + 1
        qb31 = jnp.full((_L,), qb + _NSAMPLE - 1, jnp.int32)

        @plsc.parallel_loop(0, _N // _L, unroll=4, carry=qbv - 1)
        def omq(i, om):
            siv = i * _L
            pv = bpl + siv
            px = plsc.load_gather(xq, [pv])
            py = plsc.load_gather(yq, [pv])
            pz = plsc.load_gather(zq, [pv])
            dx = px - cx
            dy = py - cy
            dz = pz - cz
            d2 = (dx * dx + dy * dy) + dz * dz
            inb = d2 < _RADIUS2
            rank = plsc.cumsum(ones, mask=inb)
            pc = plsc.all_reduce_population_count(inb)
            plsc.store_scatter(idxb, [om + rank], pv, mask=inb)
            return jnp.minimum(om + pc, qb31)

        offm1 = omq - qbv

        raw0 = idxb[pl.ds(qb, _L)]
        raw1 = idxb[pl.ds(qb + _L, _L)]
        first = plsc.load_gather(idxb, [qbv])
        sel0 = jnp.where(lanes <= offm1, raw0, first)
        sel1 = jnp.where(lhi <= offm1, raw1, first)
        idxb[pl.ds(qb, _L)] = sel0
        idxb[pl.ds(qb + _L, _L)] = sel1

    _RING = 6
    rows = (r0, r1, r2, r3, r4, r5)
    outs = (o0, o1, o2, o3, o4, o5)
    gsems = (gs0, gs1, gs2, gs3, gs4, gs5)
    osems = (os0, os1, os2, os3, os4, os5)

    def gqs(q):
        return idxb.at[pl.ds(q * _STRIDE, _NSAMPLE)]

    def outsl(q):
        return out.at[qoff + q]

    def fill(q, rbuf, obuf):
        qsp = jnp.full((_L,), q, jnp.int32)
        cx = plsc.load_gather(cxs, [qsp])
        cy = plsc.load_gather(cys, [qsp])
        cz = plsc.load_gather(czs, [qsp])
        qb = q * _STRIDE
        for h in range(2):
            g = idxb[pl.ds(qb + h * _L, _L)]
            px = plsc.load_gather(xq, [g])
            py = plsc.load_gather(yq, [g])
            pz = plsc.load_gather(zq, [g])
            obuf[0, pl.ds(h * _L, _L)] = px - cx
            obuf[1, pl.ds(h * _L, _L)] = py - cy
            obuf[2, pl.ds(h * _L, _L)] = pz - cz

        @plsc.parallel_loop(0, _C, unroll=4)
        def _(k):
            cj = (lanes + k) & (_C - 1)
            cr = cj + 3
            va = plsc.load_gather(rbuf, [lanes, cj])
            vb = plsc.load_gather(rbuf, [lhi, cj])
            plsc.store_scatter(obuf, [cr, lanes], va)
            plsc.store_scatter(obuf, [cr, lhi], vb)

    def emit(qq, b):
        pltpu.make_async_copy(featt.at[gqs(qq)], rows[b], gsems[b]).wait()

        @pl.when(qq >= _RING)
        def _():
            pltpu.make_async_copy(outs[b], outsl(qq - _RING), osems[b]).wait()

        fill(qq, rows[b], outs[b])
        pltpu.async_copy(outs[b], outsl(qq), osems[b])

    _MAIN = (_QPW // _RING) * _RING

    @pl.loop(0, _MAIN, step=_RING)
    def _(q):
        for b in range(_RING):
            qq = q + b
            scan_query(qq)

            @pl.when(qq >= _RING)
            def _():
                emit(qq - _RING, b)

            pltpu.async_copy(featt.at[gqs(qq)], rows[b], gsems[b])

    for qq in range(_MAIN, _QPW):
        b = qq % _RING
        scan_query(qq)
        emit(qq - _RING, b)
        pltpu.async_copy(featt.at[gqs(qq)], rows[b], gsems[b])

    for qq in range(_QPW - _RING, _QPW):
        emit(qq, qq % _RING)
    for qq in range(_QPW - _RING, _QPW):
        b = qq % _RING
        pltpu.make_async_copy(outs[b], outsl(qq), osems[b]).wait()


_fused = functools.partial(
    pl.kernel,
    out_type=jax.ShapeDtypeStruct((_M, 3 + _C, _NSAMPLE), jnp.float32),
    mesh=_MESH,
    compiler_params=_PARAMS,
    scratch_types=[
        pltpu.VMEM((_B * _N,), jnp.float32),
        pltpu.VMEM((_B * _N,), jnp.float32),
        pltpu.VMEM((_B * _N,), jnp.float32),
        pltpu.VMEM((_QPW,), jnp.int32),
        pltpu.VMEM((_QPW,), jnp.int32),
        pltpu.VMEM((_QPW,), jnp.int32),
        pltpu.VMEM((_QPW,), jnp.int32),
        pltpu.VMEM((_QPW,), jnp.float32),
        pltpu.VMEM((_QPW,), jnp.float32),
        pltpu.VMEM((_QPW,), jnp.float32),
        pltpu.VMEM((_QPW * _STRIDE + _L,), jnp.int32),
        pltpu.VMEM((_NSAMPLE, _C), jnp.float32),
        pltpu.VMEM((_NSAMPLE, _C), jnp.float32),
        pltpu.VMEM((_NSAMPLE, _C), jnp.float32),
        pltpu.VMEM((_NSAMPLE, _C), jnp.float32),
        pltpu.VMEM((_NSAMPLE, _C), jnp.float32),
        pltpu.VMEM((_NSAMPLE, _C), jnp.float32),
        pltpu.VMEM((3 + _C, _NSAMPLE), jnp.float32),
        pltpu.VMEM((3 + _C, _NSAMPLE), jnp.float32),
        pltpu.VMEM((3 + _C, _NSAMPLE), jnp.float32),
        pltpu.VMEM((3 + _C, _NSAMPLE), jnp.float32),
        pltpu.VMEM((3 + _C, _NSAMPLE), jnp.float32),
        pltpu.VMEM((3 + _C, _NSAMPLE), jnp.float32),
        pltpu.SemaphoreType.DMA,
    ] + [pltpu.SemaphoreType.DMA] * 12,
)(_fused_body)


def kernel(xyz, new_xyz, indices, features):
    xyzp = jnp.transpose(xyz, (2, 0, 1)).reshape(3 * _B * _N)
    ncp = jnp.transpose(new_xyz, (2, 0, 1)).reshape(3, _B * _K)
    indt = jnp.transpose(indices, (1, 0)).reshape(2 * _M)
    featt = jnp.transpose(features, (0, 2, 1)).reshape(_B * _N, _C)
    return _fused(xyzp, ncp[0], ncp[1], ncp[2], indt, featt)
